# Initial kernel scaffold; baseline (speedup 1.0000x reference)
import jax
import jax.numpy as jnp
from jax.experimental import pallas as pl


def kernel(x, edge_index, W1, b1, W2, b2):
    raise NotImplementedError("write your pallas kernel here")


# speedup vs baseline: 3.3074x; 3.3074x over previous
import functools

import jax
import jax.numpy as jnp
from jax import lax
from jax.experimental import pallas as pl
from jax.experimental.pallas import tpu as pltpu
from jax.experimental.pallas import tpu_sc as plsc

N_NODES = 10000
N_PAD = 10240
FEAT = 128
N_EDGES = 320000
K_PROP = 10
ALPHA = 0.1

CHUNK = 128
N_CHUNKS = N_EDGES // CHUNK
NSUB = 16
ROWS_PER_SUB = N_PAD // NSUB
RBLK = 128
LANES = 16


def _mlp_block(x_ref, w1_ref, b1_ref, w2_ref, b2_ref, o_ref):
    h = jnp.dot(x_ref[...], w1_ref[...], preferred_element_type=jnp.float32)
    h = jnp.maximum(h + b1_ref[...], 0.0)
    o_ref[...] = (
        jnp.dot(h, w2_ref[...], preferred_element_type=jnp.float32) + b2_ref[...]
    )


def _mlp(xp, W1, b1, W2, b2):
    BLK = 1280
    return pl.pallas_call(
        _mlp_block,
        grid=(N_PAD // BLK,),
        in_specs=[
            pl.BlockSpec((BLK, FEAT), lambda i: (i, 0)),
            pl.BlockSpec((FEAT, FEAT), lambda i: (0, 0)),
            pl.BlockSpec((1, FEAT), lambda i: (0, 0)),
            pl.BlockSpec((FEAT, FEAT), lambda i: (0, 0)),
            pl.BlockSpec((1, FEAT), lambda i: (0, 0)),
        ],
        out_specs=pl.BlockSpec((BLK, FEAT), lambda i: (i, 0)),
        out_shape=jax.ShapeDtypeStruct((N_PAD, FEAT), jnp.float32),
    )(xp, W1, b1.reshape(1, FEAT), W2, b2.reshape(1, FEAT))


def _prop_body(x0_hbm, src_hbm, dst_hbm, h_hbm,
               agg_sh, sidx_v, didx_v, rows_v, bufa_v, sem):
    sid = lax.axis_index("s")
    rbase = sid * ROWS_PER_SUB
    clo = (N_CHUNKS * sid) // NSUB
    chi = (N_CHUNKS * (sid + 1)) // NSUB

    @pl.loop(0, ROWS_PER_SUB, step=RBLK)
    def _init(r):
        pltpu.sync_copy(x0_hbm.at[pl.ds(rbase + r, RBLK)], bufa_v)
        pltpu.sync_copy(bufa_v, h_hbm.at[pl.ds(rbase + r, RBLK)])

    plsc.subcore_barrier()

    @pl.loop(0, K_PROP)
    def _iter(k):
        @pl.loop(0, ROWS_PER_SUB, step=RBLK)
        def _pz(r):
            pltpu.sync_copy(x0_hbm.at[pl.ds(rbase + r, RBLK)], bufa_v)

            @pl.loop(0, RBLK)
            def _zrow(i):
                for l in range(FEAT // LANES):
                    sl = pl.ds(l * LANES, LANES)
                    bufa_v[i, sl] = bufa_v[i, sl] * (ALPHA / (1.0 - ALPHA))

            pltpu.sync_copy(bufa_v, agg_sh.at[pl.ds(rbase + r, RBLK)])

        plsc.subcore_barrier()

        @pl.loop(clo, chi)
        def _pa(j):
            base = j * CHUNK
            pltpu.sync_copy(src_hbm.at[pl.ds(base, CHUNK)], sidx_v)
            pltpu.sync_copy(dst_hbm.at[pl.ds(base, CHUNK)], didx_v.at[0])
            pltpu.async_copy(h_hbm.at[sidx_v], rows_v, sem).wait()
            pltpu.sync_copy(rows_v, agg_sh.at[didx_v.at[0]], add=True)

        plsc.subcore_barrier()

        @pl.loop(0, ROWS_PER_SUB, step=RBLK)
        def _pb(r):
            pltpu.sync_copy(agg_sh.at[pl.ds(rbase + r, RBLK)], bufa_v)

            @pl.loop(0, RBLK)
            def _row(i):
                for l in range(FEAT // LANES):
                    sl = pl.ds(l * LANES, LANES)
                    bufa_v[i, sl] = bufa_v[i, sl] * (1.0 - ALPHA)

            pltpu.sync_copy(bufa_v, h_hbm.at[pl.ds(rbase + r, RBLK)])

        plsc.subcore_barrier()


@functools.partial(
    pl.kernel,
    out_type=jax.ShapeDtypeStruct((N_PAD, FEAT), jnp.float32),
    mesh=plsc.VectorSubcoreMesh(
        core_axis_name="c", subcore_axis_name="s", num_cores=1),
    scratch_types=[
        pltpu.VMEM_SHARED((N_PAD, FEAT), jnp.float32),
        pltpu.VMEM((CHUNK,), jnp.int32),
        pltpu.VMEM((1, CHUNK), jnp.int32),
        pltpu.VMEM((CHUNK, FEAT), jnp.float32),
        pltpu.VMEM((RBLK, FEAT), jnp.float32),
        pltpu.SemaphoreType.DMA,
    ],
)
def _propagate(x0_hbm, src_hbm, dst_hbm, h_hbm, *scratch):
    _prop_body(x0_hbm, src_hbm, dst_hbm, h_hbm, *scratch)


def kernel(x, edge_index, W1, b1, W2, b2):
    xp = jnp.concatenate(
        [x, jnp.zeros((N_PAD - N_NODES, FEAT), jnp.float32)], axis=0)
    x0 = _mlp(xp, W1, b1, W2, b2)
    hp = _propagate(x0, edge_index[0], edge_index[1])
    return hp[:N_NODES]


# speedup vs baseline: 6.5224x; 1.9720x over previous
import functools

import jax
import jax.numpy as jnp
from jax import lax
from jax.experimental import pallas as pl
from jax.experimental.pallas import tpu as pltpu
from jax.experimental.pallas import tpu_sc as plsc

N_NODES = 10000
N_PAD = 10240
FEAT = 128
N_EDGES = 320000
K_PROP = 10
ALPHA = 0.1

CHUNK = 128
N_CHUNKS = N_EDGES // CHUNK
GRP = 4
N_GROUPS = N_CHUNKS // GRP
NSUB = 16
T_MAX = 40
ROWS_PER_SUB = N_PAD // NSUB
RBLK = 128
NBLK = ROWS_PER_SUB // RBLK
LANES = 16


def _mlp_block(x_ref, w1_ref, b1_ref, w2_ref, b2_ref, o_ref, os_ref):
    h = jnp.dot(x_ref[...], w1_ref[...], preferred_element_type=jnp.float32)
    h = jnp.maximum(h + b1_ref[...], 0.0)
    o = jnp.dot(h, w2_ref[...], preferred_element_type=jnp.float32) + b2_ref[...]
    o_ref[...] = o
    os_ref[...] = o * (ALPHA / (1.0 - ALPHA))


def _mlp(xp, W1, b1, W2, b2):
    BLK = 1280
    return pl.pallas_call(
        _mlp_block,
        grid=(N_PAD // BLK,),
        in_specs=[
            pl.BlockSpec((BLK, FEAT), lambda i: (i, 0)),
            pl.BlockSpec((FEAT, FEAT), lambda i: (0, 0)),
            pl.BlockSpec((1, FEAT), lambda i: (0, 0)),
            pl.BlockSpec((FEAT, FEAT), lambda i: (0, 0)),
            pl.BlockSpec((1, FEAT), lambda i: (0, 0)),
        ],
        out_specs=[
            pl.BlockSpec((BLK, FEAT), lambda i: (i, 0)),
            pl.BlockSpec((BLK, FEAT), lambda i: (i, 0)),
        ],
        out_shape=[
            jax.ShapeDtypeStruct((N_PAD, FEAT), jnp.float32),
            jax.ShapeDtypeStruct((N_PAD, FEAT), jnp.float32),
        ],
    )(xp, W1, b1.reshape(1, FEAT), W2, b2.reshape(1, FEAT))


def _prop_body(x0_hbm, x0s_hbm, pidx_hbm, h_hbm,
               agg_sh, ib0, ib1, rows0, rows1, sem0, sem1):
    sid = lax.axis_index("s")
    rbase = sid * ROWS_PER_SUB
    rows = (rows0, rows1)
    sems = (sem0, sem1)
    ibs = (ib0, ib1)

    for rb in range(NBLK):
        sl = pl.ds(rbase + rb * RBLK, RBLK)
        pltpu.sync_copy(x0_hbm.at[sl], rows0)
        pltpu.sync_copy(rows0, h_hbm.at[sl])
    pltpu.sync_copy(x0s_hbm.at[pl.ds(rbase, ROWS_PER_SUB)],
                    agg_sh.at[pl.ds(rbase, ROWS_PER_SUB)])
    plsc.subcore_barrier()

    @pl.loop(0, K_PROP)
    def _iter(k):
        pltpu.sync_copy(pidx_hbm.at[pl.ds(sid * 2 * GRP, 2 * GRP)], ib0)
        pltpu.async_copy(h_hbm.at[ib0.at[0]], rows0, sem0)

        @pl.loop(0, T_MAX, step=2)
        def _pa(t):
            for half in range(2):
                ib = ibs[half]
                ibn = ibs[1 - half]
                g = sid + (t + half) * NSUB
                gn = sid + (t + half + 1) * NSUB
                vg = g < N_GROUPS
                vn = gn < N_GROUPS

                @pl.when(vg)
                def _do():
                    @pl.when(vn)
                    def _pf():
                        pltpu.sync_copy(
                            pidx_hbm.at[pl.ds(gn * 2 * GRP, 2 * GRP)], ibn)

                    for m in range(GRP):
                        p = m % 2
                        q = 1 - p
                        pltpu.make_async_copy(
                            h_hbm.at[ib.at[m]], rows[p], sems[p]).wait()
                        if m < GRP - 1:
                            pltpu.async_copy(
                                h_hbm.at[ib.at[m + 1]], rows[q], sems[q])
                        else:
                            @pl.when(vn)
                            def _fn():
                                pltpu.async_copy(
                                    h_hbm.at[ibn.at[0]], rows[q], sems[q])
                        pltpu.sync_copy(rows[p], agg_sh.at[ib.at[GRP + m]],
                                        add=True)

        plsc.subcore_barrier()

        pltpu.async_copy(agg_sh.at[pl.ds(rbase, RBLK)], rows0, sem0)
        for rb in range(NBLK):
            p = rb % 2
            q = 1 - p
            sl = pl.ds(rbase + rb * RBLK, RBLK)
            pltpu.make_async_copy(agg_sh.at[sl], rows[p], sems[p]).wait()
            if rb < NBLK - 1:
                sln = pl.ds(rbase + (rb + 1) * RBLK, RBLK)
                pltpu.async_copy(agg_sh.at[sln], rows[q], sems[q])

            rbuf = rows[p]

            @pl.loop(0, RBLK)
            def _row(i):
                for l in range(FEAT // LANES):
                    ls = pl.ds(l * LANES, LANES)
                    rbuf[i, ls] = rbuf[i, ls] * (1.0 - ALPHA)

            pltpu.sync_copy(rbuf, h_hbm.at[sl])
            pltpu.sync_copy(x0s_hbm.at[sl], agg_sh.at[sl])

        plsc.subcore_barrier()


@functools.partial(
    pl.kernel,
    out_type=jax.ShapeDtypeStruct((N_PAD, FEAT), jnp.float32),
    mesh=plsc.VectorSubcoreMesh(
        core_axis_name="c", subcore_axis_name="s", num_cores=1),
    scratch_types=[
        pltpu.VMEM_SHARED((N_PAD, FEAT), jnp.float32),
        pltpu.VMEM((2 * GRP, CHUNK), jnp.int32),
        pltpu.VMEM((2 * GRP, CHUNK), jnp.int32),
        pltpu.VMEM((CHUNK, FEAT), jnp.float32),
        pltpu.VMEM((CHUNK, FEAT), jnp.float32),
        pltpu.SemaphoreType.DMA,
        pltpu.SemaphoreType.DMA,
    ],
)
def _propagate(x0_hbm, x0s_hbm, pidx_hbm, h_hbm, *scratch):
    _prop_body(x0_hbm, x0s_hbm, pidx_hbm, h_hbm, *scratch)


def kernel(x, edge_index, W1, b1, W2, b2):
    xp = jnp.concatenate(
        [x, jnp.zeros((N_PAD - N_NODES, FEAT), jnp.float32)], axis=0)
    x0, x0s = _mlp(xp, W1, b1, W2, b2)
    src3 = edge_index[0].reshape(N_GROUPS, GRP, CHUNK)
    dst3 = edge_index[1].reshape(N_GROUPS, GRP, CHUNK)
    pidx = jnp.concatenate([src3, dst3], axis=1).reshape(
        N_GROUPS * 2 * GRP, CHUNK)
    hp = _propagate(x0, x0s, pidx)
    return hp[:N_NODES]


# speedup vs baseline: 11.1891x; 1.7155x over previous
import functools

import jax
import jax.numpy as jnp
from jax import lax
from jax.experimental import pallas as pl
from jax.experimental.pallas import tpu as pltpu
from jax.experimental.pallas import tpu_sc as plsc

N_NODES = 10000
N_PAD = 10240
FEAT = 128
N_EDGES = 320000
K_PROP = 10
ALPHA = 0.1

CHUNK = 128
N_CHUNKS = N_EDGES // CHUNK
GRP = 4
N_GROUPS = N_CHUNKS // GRP
NSUB = 16
NCORE = 2
NW = NCORE * NSUB
T_MAX = 20
ROWS_PER_SUB = N_PAD // NSUB
LANES = 16


def _mlp_block(x_ref, w1_ref, b1_ref, w2_ref, b2_ref, o_ref, os_ref):
    h = jnp.dot(x_ref[...], w1_ref[...], preferred_element_type=jnp.float32)
    h = jnp.maximum(h + b1_ref[...], 0.0)
    o = jnp.dot(h, w2_ref[...], preferred_element_type=jnp.float32) + b2_ref[...]
    o_ref[...] = o
    os_ref[...] = o * (0.5 * ALPHA / (1.0 - ALPHA))


def _mlp(xp, W1, b1, W2, b2):
    BLK = 1280
    return pl.pallas_call(
        _mlp_block,
        grid=(N_PAD // BLK,),
        in_specs=[
            pl.BlockSpec((BLK, FEAT), lambda i: (i, 0)),
            pl.BlockSpec((FEAT, FEAT), lambda i: (0, 0)),
            pl.BlockSpec((1, FEAT), lambda i: (0, 0)),
            pl.BlockSpec((FEAT, FEAT), lambda i: (0, 0)),
            pl.BlockSpec((1, FEAT), lambda i: (0, 0)),
        ],
        out_specs=[
            pl.BlockSpec((BLK, FEAT), lambda i: (i, 0)),
            pl.BlockSpec((BLK, FEAT), lambda i: (i, 0)),
        ],
        out_shape=[
            jax.ShapeDtypeStruct((N_PAD, FEAT), jnp.float32),
            jax.ShapeDtypeStruct((N_PAD, FEAT), jnp.float32),
        ],
    )(xp, W1, b1.reshape(1, FEAT), W2, b2.reshape(1, FEAT))


def _phase_a_body(h_hbm, x0h_hbm, pidx_hbm, aggout_hbm,
                  agg_sh, ib0, ib1, rows0, rows1, sem0, sem1):
    cid = lax.axis_index("c")
    sid = lax.axis_index("s")
    w = cid * NSUB + sid
    rbase = sid * ROWS_PER_SUB
    rows = (rows0, rows1)
    sems = (sem0, sem1)
    ibs = (ib0, ib1)

    pltpu.sync_copy(x0h_hbm.at[pl.ds(rbase, ROWS_PER_SUB)],
                    agg_sh.at[pl.ds(rbase, ROWS_PER_SUB)])
    plsc.subcore_barrier()

    pltpu.sync_copy(pidx_hbm.at[pl.ds(w * 2 * GRP, 2 * GRP)], ib0)
    pltpu.async_copy(h_hbm.at[ib0.at[0]], rows0, sem0)

    @pl.loop(0, T_MAX, step=2)
    def _pa(t):
        for half in range(2):
            ib = ibs[half]
            ibn = ibs[1 - half]
            g = w + (t + half) * NW
            gn = w + (t + half + 1) * NW
            vg = g < N_GROUPS
            vn = gn < N_GROUPS

            @pl.when(vg)
            def _do():
                @pl.when(vn)
                def _pf():
                    pltpu.sync_copy(
                        pidx_hbm.at[pl.ds(gn * 2 * GRP, 2 * GRP)], ibn)

                for m in range(GRP):
                    p = m % 2
                    q = 1 - p
                    pltpu.make_async_copy(
                        h_hbm.at[ib.at[m]], rows[p], sems[p]).wait()
                    if m < GRP - 1:
                        pltpu.async_copy(
                            h_hbm.at[ib.at[m + 1]], rows[q], sems[q])
                    else:
                        @pl.when(vn)
                        def _fn():
                            pltpu.async_copy(
                                h_hbm.at[ibn.at[0]], rows[q], sems[q])
                    pltpu.sync_copy(rows[p], agg_sh.at[ib.at[GRP + m]],
                                    add=True)

    plsc.subcore_barrier()

    pltpu.sync_copy(agg_sh.at[pl.ds(rbase, ROWS_PER_SUB)],
                    aggout_hbm.at[cid, pl.ds(rbase, ROWS_PER_SUB)])


@functools.partial(
    pl.kernel,
    out_type=jax.ShapeDtypeStruct((NCORE, N_PAD, FEAT), jnp.float32),
    mesh=plsc.VectorSubcoreMesh(
        core_axis_name="c", subcore_axis_name="s", num_cores=NCORE),
    scratch_types=[
        pltpu.VMEM_SHARED((N_PAD, FEAT), jnp.float32),
        pltpu.VMEM((2 * GRP, CHUNK), jnp.int32),
        pltpu.VMEM((2 * GRP, CHUNK), jnp.int32),
        pltpu.VMEM((CHUNK, FEAT), jnp.float32),
        pltpu.VMEM((CHUNK, FEAT), jnp.float32),
        pltpu.SemaphoreType.DMA,
        pltpu.SemaphoreType.DMA,
    ],
)
def _phase_a(h_hbm, x0h_hbm, pidx_hbm, aggout_hbm, *scratch):
    _phase_a_body(h_hbm, x0h_hbm, pidx_hbm, aggout_hbm, *scratch)


def _upd_block(a_ref, b_ref, o_ref):
    o_ref[...] = (1.0 - ALPHA) * (a_ref[0] + b_ref[0])


def _update(agg2):
    BLK = 1280
    return pl.pallas_call(
        _upd_block,
        grid=(N_PAD // BLK,),
        in_specs=[
            pl.BlockSpec((1, BLK, FEAT), lambda i: (0, i, 0)),
            pl.BlockSpec((1, BLK, FEAT), lambda i: (1, i, 0)),
        ],
        out_specs=pl.BlockSpec((BLK, FEAT), lambda i: (i, 0)),
        out_shape=jax.ShapeDtypeStruct((N_PAD, FEAT), jnp.float32),
    )(agg2, agg2)


def kernel(x, edge_index, W1, b1, W2, b2):
    xp = jnp.concatenate(
        [x, jnp.zeros((N_PAD - N_NODES, FEAT), jnp.float32)], axis=0)
    x0, x0h = _mlp(xp, W1, b1, W2, b2)
    src3 = edge_index[0].reshape(N_GROUPS, GRP, CHUNK)
    dst3 = edge_index[1].reshape(N_GROUPS, GRP, CHUNK)
    pidx = jnp.concatenate([src3, dst3], axis=1).reshape(
        N_GROUPS * 2 * GRP, CHUNK)
    h = x0
    for _ in range(K_PROP):
        agg2 = _phase_a(h, x0h, pidx)
        h = _update(agg2)
    return h[:N_NODES]
